# baseline (speedup 1.0000x reference)
import jax
import jax.numpy as jnp
from jax import lax
from jax.experimental import pallas as pl
from jax.experimental.pallas import tpu as pltpu
from jax.experimental.pallas import tpu_sc as plsc

_N_NODES = 10000
_N_EDGES = 320000
_C = 128
_N_EPAD = 327680
_K = 4
_N_SL = _N_EPAD // _K
_N_CH = _N_SL // _C
_B_E = 4096
_B_N = 2000
_TILES = 16
_N_PAD = 10240
_ROWS_PT = _N_PAD // _TILES
_CH_A_PT = _N_CH // _TILES
_CH_B_PT = _N_CH // (2 * _TILES)


def _edge_body(ef_ref, g_ref, we1_ref, be1_ref, we2_ref, wge2_ref, be2_ref,
               sl_ref, e1_ref, e2_ref, esum_ref):
    e1 = jnp.maximum(ef_ref[...] @ we1_ref[...] + be1_ref[...], 0.0)
    gterm = g_ref[...] @ wge2_ref[...] + be2_ref[...]
    e2 = jnp.maximum(e1 @ we2_ref[...] + gterm, 0.0)
    e1_ref[...] = e1
    e2_ref[...] = e2

    @pl.when(pl.program_id(0) == 0)
    def _():
        esum_ref[...] = jnp.zeros_like(esum_ref)

    rid = (sl_ref[0] * _N_SL + pl.program_id(0) * _B_E
           + lax.broadcasted_iota(jnp.int32, (_B_E, 1), 0))
    esum_ref[...] += jnp.sum(jnp.where(rid < _N_EDGES, e2, 0.0),
                             axis=0, keepdims=True)


def _edge_pass(ef, g, we1, be1, we2, wge2, be2, sl, *, interpret=False):
    n_blk = _N_SL // _B_E
    return pl.pallas_call(
        _edge_body,
        grid=(n_blk,),
        in_specs=[
            pl.BlockSpec((_B_E, 16), lambda i: (i, 0)),
            pl.BlockSpec((1, 32), lambda i: (0, 0)),
            pl.BlockSpec((16, 256), lambda i: (0, 0)),
            pl.BlockSpec((1, 256), lambda i: (0, 0)),
            pl.BlockSpec((256, 128), lambda i: (0, 0)),
            pl.BlockSpec((32, 128), lambda i: (0, 0)),
            pl.BlockSpec((1, 128), lambda i: (0, 0)),
            pl.BlockSpec(memory_space=pltpu.SMEM),
        ],
        out_specs=[
            pl.BlockSpec((_B_E, 256), lambda i: (i, 0)),
            pl.BlockSpec((_B_E, 128), lambda i: (i, 0)),
            pl.BlockSpec((1, 128), lambda i: (0, 0)),
        ],
        out_shape=[
            jax.ShapeDtypeStruct((_N_SL, 256), jnp.float32),
            jax.ShapeDtypeStruct((_N_SL, 128), jnp.float32),
            jax.ShapeDtypeStruct((1, 128), jnp.float32),
        ],
        interpret=interpret,
    )(ef, g, we1, be1, we2, wge2, be2, sl)


def _sc_agg_body(e1_hbm, e2_hbm, recv_hbm, zeros_hbm, zeros1_hbm, ones_hbm,
                 agg1_out, cnt_out, agg2_out,
                 acc, cntacc, upd, idxb, ones_v, sem0, sem1):
    cid = lax.axis_index("c")
    sid = lax.axis_index("s")
    r0 = sid * _ROWS_PT
    col0 = cid * 128

    pltpu.sync_copy(zeros_hbm.at[pl.ds(r0, _ROWS_PT), :],
                    acc.at[pl.ds(r0, _ROWS_PT), :])

    @pl.when(cid == 0)
    def _():
        pltpu.sync_copy(zeros1_hbm.at[pl.ds(r0, _ROWS_PT)],
                        cntacc.at[pl.ds(r0, _ROWS_PT)])

    pltpu.sync_copy(ones_hbm, ones_v)
    plsc.subcore_barrier()

    def _run_phase(src_slice, idx_row0, idx_rows, n_chunks, rowfn,
                   with_counts):
        def _start(r, b):
            pltpu.async_copy(src_slice(idx_row0 + r), upd.at[b],
                             sem0 if b == 0 else sem1)

        def _wait(b):
            pltpu.make_async_copy(src_slice(0), upd.at[b],
                                  sem0 if b == 0 else sem1).wait()

        def _scatter(r, b):
            pltpu.sync_copy(upd.at[b], acc.at[idxb.at[r]], add=True)
            if with_counts:
                @pl.when(cid == 0)
                def _():
                    pltpu.sync_copy(ones_v, cntacc.at[idxb.at[r]], add=True)

        pltpu.sync_copy(recv_hbm.at[pl.ds(idx_row0, idx_rows), :],
                        idxb.at[pl.ds(0, idx_rows), :])
        _start(rowfn(0), 0)

        def _pair(i, carry):
            k0 = 2 * i
            k1 = k0 + 1
            _start(rowfn(k1), 1)
            _wait(0)
            _scatter(rowfn(k0), 0)

            @pl.when(k0 + 2 < n_chunks)
            def _():
                _start(rowfn(k0 + 2), 0)

            _wait(1)
            _scatter(rowfn(k1), 1)
            return carry

        lax.fori_loop(0, n_chunks // 2, _pair, 0)

    _run_phase(
        lambda ch: e1_hbm.at[pl.ds(ch * _C, _C), pl.ds(col0, 128)],
        sid * _CH_A_PT, _CH_A_PT, _CH_A_PT, lambda k: k, True)
    plsc.subcore_barrier()

    pltpu.sync_copy(acc.at[pl.ds(r0, _ROWS_PT), :],
                    agg1_out.at[pl.ds(r0, _ROWS_PT), pl.ds(col0, 128)])

    @pl.when(cid == 0)
    def _():
        pltpu.sync_copy(cntacc.at[pl.ds(r0, _ROWS_PT)],
                        cnt_out.at[pl.ds(r0, _ROWS_PT)])

    pltpu.sync_copy(zeros_hbm.at[pl.ds(r0, _ROWS_PT), :],
                    acc.at[pl.ds(r0, _ROWS_PT), :])
    plsc.subcore_barrier()

    _run_phase(
        lambda ch: e2_hbm.at[pl.ds(ch * _C, _C), :],
        sid * _CH_A_PT, _CH_A_PT, _CH_B_PT, lambda k: 2 * k + cid, False)
    plsc.subcore_barrier()
    pltpu.sync_copy(acc.at[pl.ds(r0, _ROWS_PT), :],
                    agg2_out.at[cid, pl.ds(r0, _ROWS_PT), :])


def _sc_aggregate(e1, e2, recv2, zeros_n, zeros1, ones_h):
    agg = pl.kernel(
        _sc_agg_body,
        cost_estimate=pl.CostEstimate(
            flops=0, bytes_accessed=250_000_000, transcendentals=0),
        out_type=[
            jax.ShapeDtypeStruct((_N_PAD, 256), jnp.float32),
            jax.ShapeDtypeStruct((_N_PAD,), jnp.float32),
            jax.ShapeDtypeStruct((2, _N_PAD, 128), jnp.float32),
        ],
        mesh=plsc.VectorSubcoreMesh(core_axis_name="c", subcore_axis_name="s"),
        scratch_types=[
            pltpu.VMEM_SHARED((_N_PAD, 128), jnp.float32),
            pltpu.VMEM_SHARED((_N_PAD,), jnp.float32),
            pltpu.VMEM((2, _C, 128), jnp.float32),
            pltpu.VMEM((_CH_A_PT, _C), jnp.int32),
            pltpu.VMEM((_C,), jnp.float32),
            pltpu.SemaphoreType.DMA,
            pltpu.SemaphoreType.DMA,
        ],
    )
    return agg(e1, e2, recv2, zeros_n, zeros1, ones_h)


def _node_body(*refs):
    (nf_ref, g_ref, wn1_ref, win1_ref, bn1_ref, wn2_ref, win2_ref,
     wgn2_ref, bn2_ref) = refs[:9]
    a1_refs = refs[9:9 + _K]
    cnt_refs = refs[9 + _K:9 + 2 * _K]
    p_refs = refs[9 + 2 * _K:9 + 4 * _K]
    nsum_ref = refs[-1]

    cnt = cnt_refs[0][...]
    for r in cnt_refs[1:]:
        cnt = cnt + r[...]
    cnt = jnp.maximum(cnt, 1.0)
    a1 = a1_refs[0][...]
    for r in a1_refs[1:]:
        a1 = a1 + r[...]
    p = p_refs[0][...]
    for r in p_refs[1:]:
        p = p + r[...]
    agg1 = a1 / cnt
    agg2 = p / cnt
    n1 = jnp.maximum(nf_ref[...] @ wn1_ref[...] + agg1 @ win1_ref[...]
                     + bn1_ref[...], 0.0)
    gterm = g_ref[...] @ wgn2_ref[...] + bn2_ref[...]
    n2 = jnp.maximum(n1 @ wn2_ref[...] + agg2 @ win2_ref[...] + gterm, 0.0)

    @pl.when(pl.program_id(0) == 0)
    def _():
        nsum_ref[...] = jnp.zeros_like(nsum_ref)

    nsum_ref[...] += jnp.sum(n2, axis=0, keepdims=True)


def _node_pass(nf, g, wn1, win1, bn1, wn2, win2, wgn2, bn2,
               a1s, cnts, ps, *, interpret=False):
    n_blk = _N_NODES // _B_N
    node_blk = lambda w: pl.BlockSpec((_B_N, w), lambda i: (i, 0))
    const_blk = lambda r, c: pl.BlockSpec((r, c), lambda i: (0, 0))
    return pl.pallas_call(
        _node_body,
        grid=(n_blk,),
        in_specs=(
            [node_blk(128), const_blk(1, 32),
             const_blk(128, 256), const_blk(256, 256), const_blk(1, 256),
             const_blk(256, 128), const_blk(128, 128), const_blk(32, 128),
             const_blk(1, 128)]
            + [node_blk(256)] * _K
            + [node_blk(1)] * _K
            + [node_blk(128)] * (2 * _K)
        ),
        out_specs=pl.BlockSpec((1, 128), lambda i: (0, 0)),
        out_shape=jax.ShapeDtypeStruct((1, 128), jnp.float32),
        interpret=interpret,
    )(nf, g, wn1, win1, bn1, wn2, win2, wgn2, bn2, *a1s, *cnts, *ps)


def _head_body(*refs):
    (nsum_ref, g_ref, wgn_ref, wge_ref, wgg_ref, bg_ref,
     wm_ref, bm_ref, wl_ref, bl_ref) = refs[:10]
    es_refs = refs[10:10 + _K]
    mean_ref, logstd_ref = refs[-2:]
    esum = es_refs[0][...]
    for r in es_refs[1:]:
        esum = esum + r[...]
    u = (nsum_ref[...] * (1.0 / _N_NODES)) @ wgn_ref[...] \
        + (esum * (1.0 / _N_EDGES)) @ wge_ref[...] \
        + g_ref[...] @ wgg_ref[...] + bg_ref[...]
    gv = jnp.maximum(u, 0.0)
    mean_ref[...] = gv @ wm_ref[...] + bm_ref[...]
    logstd_ref[...] = jnp.clip(gv @ wl_ref[...] + bl_ref[...], -20.0, 2.0)


def _head_pass(nsum, g, wgn, wge, wgg, bg, wm, bm, wl, bl, esums,
               *, interpret=False):
    return pl.pallas_call(
        _head_body,
        out_shape=[
            jax.ShapeDtypeStruct((1, 8), jnp.float32),
            jax.ShapeDtypeStruct((1, 8), jnp.float32),
        ],
        interpret=interpret,
    )(nsum, g, wgn, wge, wgg, bg, wm, bm, wl, bl, *esums)


def kernel(node_features, edge_features, global_features, edge_index,
           W_e1, b_e1, W_n1, W_in1, b_n1,
           W_e2, W_ge2, b_e2,
           W_n2, W_in2, W_gn2, b_n2,
           W_gn, W_gedge, W_gg, b_g,
           W_mean, b_mean, W_logstd, b_logstd):
    n_pad_e = _N_EPAD - _N_EDGES
    recv = edge_index[1].astype(jnp.int32)
    pad_idx = _N_NODES + (jnp.arange(n_pad_e, dtype=jnp.int32)
                          % (_N_PAD - _N_NODES))
    recv_pad = jnp.concatenate([recv, pad_idx])
    ef_pad = jnp.concatenate(
        [edge_features, jnp.zeros((n_pad_e, 16), jnp.float32)], axis=0)
    zeros_n = jnp.zeros((_N_PAD, 128), jnp.float32)
    zeros1 = jnp.zeros((_N_PAD,), jnp.float32)
    ones_h = jnp.ones((_C,), jnp.float32)
    be1 = b_e1.reshape(1, -1)
    be2 = b_e2.reshape(1, -1)

    a1s, cnts, ps, esums = [], [], [], []
    for h in range(_K):
        ef_h = lax.slice_in_dim(ef_pad, h * _N_SL, (h + 1) * _N_SL)
        recv_h = lax.slice_in_dim(recv_pad, h * _N_SL,
                                  (h + 1) * _N_SL).reshape(_N_CH, _C)
        e1, e2, esum = _edge_pass(
            ef_h, global_features, W_e1, be1, W_e2, W_ge2, be2,
            jnp.array([h], jnp.int32))
        agg1s, cnt1, agg2p = _sc_aggregate(
            e1, e2, recv_h, zeros_n, zeros1, ones_h)
        a1s.append(agg1s)
        cnts.append(cnt1.reshape(_N_PAD, 1))
        ps.extend([agg2p[0], agg2p[1]])
        esums.append(esum)

    nsum = _node_pass(
        node_features, global_features, W_n1, W_in1, b_n1.reshape(1, -1),
        W_n2, W_in2, W_gn2, b_n2.reshape(1, -1), a1s, cnts, ps)
    return _head_pass(
        nsum, global_features, W_gn, W_gedge, W_gg, b_g.reshape(1, -1),
        W_mean, b_mean, W_logstd, b_logstd.reshape(1, -1), esums)


# speedup vs baseline: 1.0948x; 1.0948x over previous
import jax
import jax.numpy as jnp
from jax import lax
from jax.experimental import pallas as pl
from jax.experimental.pallas import tpu as pltpu
from jax.experimental.pallas import tpu_sc as plsc

_N_NODES = 10000
_N_EDGES = 320000
_C = 128
_N_EPAD = 327680
_K = 2
_N_SL = _N_EPAD // _K
_N_CH = _N_SL // _C
_B_E = 4096
_B_N = 2000
_TILES = 16
_N_PAD = 10240
_ROWS_PT = _N_PAD // _TILES
_CH_A_PT = _N_CH // _TILES
_CH_B_PT = _N_CH // (2 * _TILES)


def _edge_body(ef_ref, g_ref, we1_ref, be1_ref, we2_ref, wge2_ref, be2_ref,
               sl_ref, e1_ref, e2_ref, esum_ref):
    e1 = jnp.maximum(ef_ref[...] @ we1_ref[...] + be1_ref[...], 0.0)
    gterm = g_ref[...] @ wge2_ref[...] + be2_ref[...]
    e2 = jnp.maximum(e1 @ we2_ref[...] + gterm, 0.0)
    e1_ref[...] = e1
    e2_ref[...] = e2

    @pl.when(pl.program_id(0) == 0)
    def _():
        esum_ref[...] = jnp.zeros_like(esum_ref)

    rid = (sl_ref[0] * _N_SL + pl.program_id(0) * _B_E
           + lax.broadcasted_iota(jnp.int32, (_B_E, 1), 0))
    esum_ref[...] += jnp.sum(jnp.where(rid < _N_EDGES, e2, 0.0),
                             axis=0, keepdims=True)


def _edge_pass(ef, g, we1, be1, we2, wge2, be2, sl, *, interpret=False):
    n_blk = _N_SL // _B_E
    return pl.pallas_call(
        _edge_body,
        grid=(n_blk,),
        in_specs=[
            pl.BlockSpec((_B_E, 16), lambda i: (i, 0)),
            pl.BlockSpec((1, 32), lambda i: (0, 0)),
            pl.BlockSpec((16, 256), lambda i: (0, 0)),
            pl.BlockSpec((1, 256), lambda i: (0, 0)),
            pl.BlockSpec((256, 128), lambda i: (0, 0)),
            pl.BlockSpec((32, 128), lambda i: (0, 0)),
            pl.BlockSpec((1, 128), lambda i: (0, 0)),
            pl.BlockSpec(memory_space=pltpu.SMEM),
        ],
        out_specs=[
            pl.BlockSpec((_B_E, 256), lambda i: (i, 0)),
            pl.BlockSpec((_B_E, 128), lambda i: (i, 0)),
            pl.BlockSpec((1, 128), lambda i: (0, 0)),
        ],
        out_shape=[
            jax.ShapeDtypeStruct((_N_SL, 256), jnp.float32),
            jax.ShapeDtypeStruct((_N_SL, 128), jnp.float32),
            jax.ShapeDtypeStruct((1, 128), jnp.float32),
        ],
        interpret=interpret,
    )(ef, g, we1, be1, we2, wge2, be2, sl)


def _sc_agg_body(e1_hbm, e2_hbm, recv_hbm, zeros_hbm, zeros1_hbm, ones_hbm,
                 agg1_out, cnt_out, agg2_out,
                 acc, cntacc, upd, idxb, ones_v, sem0, sem1):
    cid = lax.axis_index("c")
    sid = lax.axis_index("s")
    r0 = sid * _ROWS_PT
    col0 = cid * 128

    pltpu.sync_copy(zeros_hbm.at[pl.ds(r0, _ROWS_PT), :],
                    acc.at[pl.ds(r0, _ROWS_PT), :])

    pltpu.sync_copy(zeros1_hbm.at[pl.ds(r0, _ROWS_PT)],
                    cntacc.at[pl.ds(r0, _ROWS_PT)])
    pltpu.sync_copy(ones_hbm, ones_v)
    plsc.subcore_barrier()

    def _run_phase(src_slice, idx_row0, idx_rows, n_chunks, rowfn,
                   with_counts):
        def _start(r, b):
            pltpu.async_copy(src_slice(idx_row0 + r), upd.at[b],
                             sem0 if b == 0 else sem1)

        def _wait(b):
            pltpu.make_async_copy(src_slice(0), upd.at[b],
                                  sem0 if b == 0 else sem1).wait()

        def _scatter(r, b):
            pltpu.sync_copy(upd.at[b], acc.at[idxb.at[r]], add=True)
            if with_counts:
                pltpu.sync_copy(ones_v, cntacc.at[idxb.at[r]], add=True)

        pltpu.sync_copy(recv_hbm.at[pl.ds(idx_row0, idx_rows), :],
                        idxb.at[pl.ds(0, idx_rows), :])
        _start(rowfn(0), 0)

        def _pair(i, carry):
            k0 = 2 * i
            k1 = k0 + 1
            _start(rowfn(k1), 1)
            _wait(0)
            _scatter(rowfn(k0), 0)

            @pl.when(k0 + 2 < n_chunks)
            def _():
                _start(rowfn(k0 + 2), 0)

            _wait(1)
            _scatter(rowfn(k1), 1)
            return carry

        lax.fori_loop(0, n_chunks // 2, _pair, 0)

    _run_phase(
        lambda ch: e1_hbm.at[pl.ds(ch * _C, _C), pl.ds(col0, 128)],
        sid * _CH_A_PT, _CH_A_PT, _CH_A_PT, lambda k: k, False)
    plsc.subcore_barrier()

    pltpu.sync_copy(acc.at[pl.ds(r0, _ROWS_PT), :],
                    agg1_out.at[pl.ds(r0, _ROWS_PT), pl.ds(col0, 128)])

    pltpu.sync_copy(zeros_hbm.at[pl.ds(r0, _ROWS_PT), :],
                    acc.at[pl.ds(r0, _ROWS_PT), :])
    plsc.subcore_barrier()

    _run_phase(
        lambda ch: e2_hbm.at[pl.ds(ch * _C, _C), :],
        sid * _CH_A_PT, _CH_A_PT, _CH_B_PT, lambda k: 2 * k + cid, True)
    plsc.subcore_barrier()
    pltpu.sync_copy(acc.at[pl.ds(r0, _ROWS_PT), :],
                    agg2_out.at[cid, pl.ds(r0, _ROWS_PT), :])
    pltpu.sync_copy(cntacc.at[pl.ds(r0, _ROWS_PT)],
                    cnt_out.at[cid, pl.ds(r0, _ROWS_PT)])


def _sc_aggregate(e1, e2, recv2, zeros_n, zeros1, ones_h):
    agg = pl.kernel(
        _sc_agg_body,
        cost_estimate=pl.CostEstimate(
            flops=0, bytes_accessed=250_000_000, transcendentals=0),
        out_type=[
            jax.ShapeDtypeStruct((_N_PAD, 256), jnp.float32),
            jax.ShapeDtypeStruct((2, _N_PAD), jnp.float32),
            jax.ShapeDtypeStruct((2, _N_PAD, 128), jnp.float32),
        ],
        mesh=plsc.VectorSubcoreMesh(core_axis_name="c", subcore_axis_name="s"),
        scratch_types=[
            pltpu.VMEM_SHARED((_N_PAD, 128), jnp.float32),
            pltpu.VMEM_SHARED((_N_PAD,), jnp.float32),
            pltpu.VMEM((2, _C, 128), jnp.float32),
            pltpu.VMEM((_CH_A_PT, _C), jnp.int32),
            pltpu.VMEM((_C,), jnp.float32),
            pltpu.SemaphoreType.DMA,
            pltpu.SemaphoreType.DMA,
        ],
    )
    return agg(e1, e2, recv2, zeros_n, zeros1, ones_h)


def _node_body(*refs):
    (nf_ref, g_ref, wn1_ref, win1_ref, bn1_ref, wn2_ref, win2_ref,
     wgn2_ref, bn2_ref) = refs[:9]
    a1_refs = refs[9:9 + _K]
    cnt_refs = refs[9 + _K:9 + 3 * _K]
    p_refs = refs[9 + 3 * _K:9 + 5 * _K]
    nsum_ref = refs[-1]

    cnt = cnt_refs[0][...]
    for r in cnt_refs[1:]:
        cnt = cnt + r[...]
    cnt = jnp.maximum(cnt, 1.0)
    a1 = a1_refs[0][...]
    for r in a1_refs[1:]:
        a1 = a1 + r[...]
    p = p_refs[0][...]
    for r in p_refs[1:]:
        p = p + r[...]
    agg1 = a1 / cnt
    agg2 = p / cnt
    n1 = jnp.maximum(nf_ref[...] @ wn1_ref[...] + agg1 @ win1_ref[...]
                     + bn1_ref[...], 0.0)
    gterm = g_ref[...] @ wgn2_ref[...] + bn2_ref[...]
    n2 = jnp.maximum(n1 @ wn2_ref[...] + agg2 @ win2_ref[...] + gterm, 0.0)

    @pl.when(pl.program_id(0) == 0)
    def _():
        nsum_ref[...] = jnp.zeros_like(nsum_ref)

    nsum_ref[...] += jnp.sum(n2, axis=0, keepdims=True)


def _node_pass(nf, g, wn1, win1, bn1, wn2, win2, wgn2, bn2,
               a1s, cnts, ps, *, interpret=False):
    n_blk = _N_NODES // _B_N
    node_blk = lambda w: pl.BlockSpec((_B_N, w), lambda i: (i, 0))
    const_blk = lambda r, c: pl.BlockSpec((r, c), lambda i: (0, 0))
    return pl.pallas_call(
        _node_body,
        grid=(n_blk,),
        in_specs=(
            [node_blk(128), const_blk(1, 32),
             const_blk(128, 256), const_blk(256, 256), const_blk(1, 256),
             const_blk(256, 128), const_blk(128, 128), const_blk(32, 128),
             const_blk(1, 128)]
            + [node_blk(256)] * _K
            + [node_blk(1)] * (2 * _K)
            + [node_blk(128)] * (2 * _K)
        ),
        out_specs=pl.BlockSpec((1, 128), lambda i: (0, 0)),
        out_shape=jax.ShapeDtypeStruct((1, 128), jnp.float32),
        interpret=interpret,
    )(nf, g, wn1, win1, bn1, wn2, win2, wgn2, bn2, *a1s, *cnts, *ps)


def _head_body(*refs):
    (nsum_ref, g_ref, wgn_ref, wge_ref, wgg_ref, bg_ref,
     wm_ref, bm_ref, wl_ref, bl_ref) = refs[:10]
    es_refs = refs[10:10 + _K]
    mean_ref, logstd_ref = refs[-2:]
    esum = es_refs[0][...]
    for r in es_refs[1:]:
        esum = esum + r[...]
    u = (nsum_ref[...] * (1.0 / _N_NODES)) @ wgn_ref[...] \
        + (esum * (1.0 / _N_EDGES)) @ wge_ref[...] \
        + g_ref[...] @ wgg_ref[...] + bg_ref[...]
    gv = jnp.maximum(u, 0.0)
    mean_ref[...] = gv @ wm_ref[...] + bm_ref[...]
    logstd_ref[...] = jnp.clip(gv @ wl_ref[...] + bl_ref[...], -20.0, 2.0)


def _head_pass(nsum, g, wgn, wge, wgg, bg, wm, bm, wl, bl, esums,
               *, interpret=False):
    return pl.pallas_call(
        _head_body,
        out_shape=[
            jax.ShapeDtypeStruct((1, 8), jnp.float32),
            jax.ShapeDtypeStruct((1, 8), jnp.float32),
        ],
        interpret=interpret,
    )(nsum, g, wgn, wge, wgg, bg, wm, bm, wl, bl, *esums)


def kernel(node_features, edge_features, global_features, edge_index,
           W_e1, b_e1, W_n1, W_in1, b_n1,
           W_e2, W_ge2, b_e2,
           W_n2, W_in2, W_gn2, b_n2,
           W_gn, W_gedge, W_gg, b_g,
           W_mean, b_mean, W_logstd, b_logstd):
    n_pad_e = _N_EPAD - _N_EDGES
    recv = edge_index[1].astype(jnp.int32)
    pad_idx = _N_NODES + (jnp.arange(n_pad_e, dtype=jnp.int32)
                          % (_N_PAD - _N_NODES))
    recv_pad = jnp.concatenate([recv, pad_idx])
    ef_pad = jnp.concatenate(
        [edge_features, jnp.zeros((n_pad_e, 16), jnp.float32)], axis=0)
    zeros_n = jnp.zeros((_N_PAD, 128), jnp.float32)
    zeros1 = jnp.zeros((_N_PAD,), jnp.float32)
    ones_h = jnp.ones((_C,), jnp.float32)
    be1 = b_e1.reshape(1, -1)
    be2 = b_e2.reshape(1, -1)

    a1s, cnts, ps, esums = [], [], [], []
    for h in range(_K):
        ef_h = lax.slice_in_dim(ef_pad, h * _N_SL, (h + 1) * _N_SL)
        recv_h = lax.slice_in_dim(recv_pad, h * _N_SL,
                                  (h + 1) * _N_SL).reshape(_N_CH, _C)
        e1, e2, esum = _edge_pass(
            ef_h, global_features, W_e1, be1, W_e2, W_ge2, be2,
            jnp.array([h], jnp.int32))
        agg1s, cnt1, agg2p = _sc_aggregate(
            e1, e2, recv_h, zeros_n, zeros1, ones_h)
        a1s.append(agg1s)
        cnts.extend([cnt1[0].reshape(_N_PAD, 1), cnt1[1].reshape(_N_PAD, 1)])
        ps.extend([agg2p[0], agg2p[1]])
        esums.append(esum)

    nsum = _node_pass(
        node_features, global_features, W_n1, W_in1, b_n1.reshape(1, -1),
        W_n2, W_in2, W_gn2, b_n2.reshape(1, -1), a1s, cnts, ps)
    return _head_pass(
        nsum, global_features, W_gn, W_gedge, W_gg, b_g.reshape(1, -1),
        W_mean, b_mean, W_logstd, b_logstd.reshape(1, -1), esums)


# speedup vs baseline: 1.1157x; 1.0190x over previous
import jax
import jax.numpy as jnp
from jax import lax
from jax.experimental import pallas as pl
from jax.experimental.pallas import tpu as pltpu
from jax.experimental.pallas import tpu_sc as plsc

_N_NODES = 10000
_N_EDGES = 320000
_C = 128
_N_EPAD = 327680
_K = 1
_N_SL = _N_EPAD // _K
_N_CH = _N_SL // _C
_B_E = 2560
_EB_TOT = _N_EDGES // _B_E
_EB_SL = _N_SL // _B_E
_B_N = 2000
_TILES = 16
_N_PAD = 10240
_ROWS_PT = _N_PAD // _TILES
_CH_A_PT = _N_CH // _TILES
_IDXB = 80
_IDX_BLKS = _CH_A_PT // _IDXB


def _edge_body(ef_ref, g_ref, we1_ref, be1_ref, we2_ref, wge2_ref, be2_ref,
               sl_ref, e1_ref, e2_ref, esum_ref):
    e1 = jnp.maximum(ef_ref[...] @ we1_ref[...] + be1_ref[...], 0.0)
    gterm = g_ref[...] @ wge2_ref[...] + be2_ref[...]
    e2 = jnp.maximum(e1 @ we2_ref[...] + gterm, 0.0)
    e1_ref[...] = e1
    e2_ref[...] = e2

    @pl.when(pl.program_id(0) == 0)
    def _():
        esum_ref[...] = jnp.zeros_like(esum_ref)

    rid = (sl_ref[0] * _N_SL + pl.program_id(0) * _B_E
           + lax.broadcasted_iota(jnp.int32, (_B_E, 1), 0))
    esum_ref[...] += jnp.sum(jnp.where(rid < _N_EDGES, e2, 0.0),
                             axis=0, keepdims=True)


def _edge_pass(ef, g, we1, be1, we2, wge2, be2, sl, h, *, interpret=False):
    base = h * _EB_SL
    return pl.pallas_call(
        _edge_body,
        grid=(_EB_SL,),
        in_specs=[
            pl.BlockSpec((_B_E, 16),
                         lambda i: (jnp.minimum(base + i, _EB_TOT - 1), 0)),
            pl.BlockSpec((1, 32), lambda i: (0, 0)),
            pl.BlockSpec((16, 256), lambda i: (0, 0)),
            pl.BlockSpec((1, 256), lambda i: (0, 0)),
            pl.BlockSpec((256, 128), lambda i: (0, 0)),
            pl.BlockSpec((32, 128), lambda i: (0, 0)),
            pl.BlockSpec((1, 128), lambda i: (0, 0)),
            pl.BlockSpec(memory_space=pltpu.SMEM),
        ],
        out_specs=[
            pl.BlockSpec((_B_E, 256), lambda i: (i, 0)),
            pl.BlockSpec((_B_E, 128), lambda i: (i, 0)),
            pl.BlockSpec((1, 128), lambda i: (0, 0)),
        ],
        out_shape=[
            jax.ShapeDtypeStruct((_N_SL, 256), jnp.float32),
            jax.ShapeDtypeStruct((_N_SL, 128), jnp.float32),
            jax.ShapeDtypeStruct((1, 128), jnp.float32),
        ],
        interpret=interpret,
    )(ef, g, we1, be1, we2, wge2, be2, sl)


def _sc_agg_body(e1_hbm, e2_hbm, recv_hbm, zeros_hbm, zeros1_hbm, ones_hbm,
                 agg1_out, cnt_out, agg2_out,
                 acc, cntacc, upd, idxb, ones_v, sem0, sem1):
    cid = lax.axis_index("c")
    sid = lax.axis_index("s")
    r0 = sid * _ROWS_PT
    col0 = cid * 128

    pltpu.sync_copy(zeros_hbm.at[pl.ds(r0, _ROWS_PT), :],
                    acc.at[pl.ds(r0, _ROWS_PT), :])
    pltpu.sync_copy(zeros1_hbm.at[pl.ds(r0, _ROWS_PT)],
                    cntacc.at[pl.ds(r0, _ROWS_PT)])
    pltpu.sync_copy(ones_hbm, ones_v)
    plsc.subcore_barrier()

    def _run_phase(src_slice, n_chunks, rowfn, with_counts):
        for blk in range(_IDX_BLKS):
            row0 = sid * _CH_A_PT + blk * _IDXB

            def _start(r, b):
                pltpu.async_copy(src_slice(row0 + r), upd.at[b],
                                 sem0 if b == 0 else sem1)

            def _wait(b):
                pltpu.make_async_copy(src_slice(0), upd.at[b],
                                      sem0 if b == 0 else sem1).wait()

            def _scatter(r, b):
                pltpu.sync_copy(upd.at[b], acc.at[idxb.at[r]], add=True)
                if with_counts:
                    pltpu.sync_copy(ones_v, cntacc.at[idxb.at[r]], add=True)

            pltpu.sync_copy(recv_hbm.at[pl.ds(row0, _IDXB), :], idxb)
            _start(rowfn(0), 0)

            def _pair(i, carry):
                k0 = 2 * i
                k1 = k0 + 1
                _start(rowfn(k1), 1)
                _wait(0)
                _scatter(rowfn(k0), 0)

                @pl.when(k0 + 2 < n_chunks)
                def _():
                    _start(rowfn(k0 + 2), 0)

                _wait(1)
                _scatter(rowfn(k1), 1)
                return carry

            lax.fori_loop(0, n_chunks // 2, _pair, 0)

    _run_phase(
        lambda ch: e1_hbm.at[pl.ds(ch * _C, _C), pl.ds(col0, 128)],
        _IDXB, lambda k: k, False)
    plsc.subcore_barrier()

    pltpu.sync_copy(acc.at[pl.ds(r0, _ROWS_PT), :],
                    agg1_out.at[pl.ds(r0, _ROWS_PT), pl.ds(col0, 128)])
    pltpu.sync_copy(zeros_hbm.at[pl.ds(r0, _ROWS_PT), :],
                    acc.at[pl.ds(r0, _ROWS_PT), :])
    plsc.subcore_barrier()

    _run_phase(
        lambda ch: e2_hbm.at[pl.ds(ch * _C, _C), :],
        _IDXB // 2, lambda k: 2 * k + cid, True)
    plsc.subcore_barrier()
    pltpu.sync_copy(acc.at[pl.ds(r0, _ROWS_PT), :],
                    agg2_out.at[cid, pl.ds(r0, _ROWS_PT), :])
    pltpu.sync_copy(cntacc.at[pl.ds(r0, _ROWS_PT)],
                    cnt_out.at[pl.ds(cid * _N_PAD + r0, _ROWS_PT)])


def _sc_aggregate(e1, e2, recv2, zeros_n, zeros1, ones_h):
    agg = pl.kernel(
        _sc_agg_body,
        cost_estimate=pl.CostEstimate(
            flops=0, bytes_accessed=1_000_000_000 // _K, transcendentals=0),
        out_type=[
            jax.ShapeDtypeStruct((_N_PAD, 256), jnp.float32),
            jax.ShapeDtypeStruct((2 * _N_PAD,), jnp.float32),
            jax.ShapeDtypeStruct((2, _N_PAD, 128), jnp.float32),
        ],
        mesh=plsc.VectorSubcoreMesh(core_axis_name="c", subcore_axis_name="s"),
        scratch_types=[
            pltpu.VMEM_SHARED((_N_PAD, 128), jnp.float32),
            pltpu.VMEM_SHARED((_N_PAD,), jnp.float32),
            pltpu.VMEM((2, _C, 128), jnp.float32),
            pltpu.VMEM((_IDXB, _C), jnp.int32),
            pltpu.VMEM((_C,), jnp.float32),
            pltpu.SemaphoreType.DMA,
            pltpu.SemaphoreType.DMA,
        ],
    )
    return agg(e1, e2, recv2, zeros_n, zeros1, ones_h)


def _node_body(*refs):
    (nf_ref, g_ref, wn1_ref, win1_ref, bn1_ref, wn2_ref, win2_ref,
     wgn2_ref, bn2_ref) = refs[:9]
    a1_refs = refs[9:9 + _K]
    cnt_refs = refs[9 + _K:9 + 3 * _K]
    p_refs = refs[9 + 3 * _K:9 + 5 * _K]
    nsum_ref = refs[-1]

    cnt = cnt_refs[0][0]
    for r in cnt_refs[1:]:
        cnt = cnt + r[0]
    cnt = jnp.maximum(cnt, 1.0)
    a1 = a1_refs[0][...]
    for r in a1_refs[1:]:
        a1 = a1 + r[...]
    p = p_refs[0][0]
    for r in p_refs[1:]:
        p = p + r[0]
    agg1 = a1 / cnt
    agg2 = p / cnt
    n1 = jnp.maximum(nf_ref[...] @ wn1_ref[...] + agg1 @ win1_ref[...]
                     + bn1_ref[...], 0.0)
    gterm = g_ref[...] @ wgn2_ref[...] + bn2_ref[...]
    n2 = jnp.maximum(n1 @ wn2_ref[...] + agg2 @ win2_ref[...] + gterm, 0.0)

    @pl.when(pl.program_id(0) == 0)
    def _():
        nsum_ref[...] = jnp.zeros_like(nsum_ref)

    nsum_ref[...] += jnp.sum(n2, axis=0, keepdims=True)


def _node_pass(nf, g, wn1, win1, bn1, wn2, win2, wgn2, bn2,
               a1s, cnts, ps, *, interpret=False):
    n_blk = _N_NODES // _B_N
    node_blk = lambda w: pl.BlockSpec((_B_N, w), lambda i: (i, 0))
    const_blk = lambda r, c: pl.BlockSpec((r, c), lambda i: (0, 0))
    core_blk = lambda c, w: pl.BlockSpec((1, _B_N, w),
                                         lambda i, c=c: (c, i, 0))
    cnt_specs, cnt_args = [], []
    for arr in cnts:
        for c in range(2):
            cnt_specs.append(core_blk(c, 1))
            cnt_args.append(arr)
    p_specs, p_args = [], []
    for arr in ps:
        for c in range(2):
            p_specs.append(core_blk(c, 128))
            p_args.append(arr)
    return pl.pallas_call(
        _node_body,
        grid=(n_blk,),
        in_specs=(
            [node_blk(128), const_blk(1, 32),
             const_blk(128, 256), const_blk(256, 256), const_blk(1, 256),
             const_blk(256, 128), const_blk(128, 128), const_blk(32, 128),
             const_blk(1, 128)]
            + [node_blk(256)] * _K
            + cnt_specs + p_specs
        ),
        out_specs=pl.BlockSpec((1, 128), lambda i: (0, 0)),
        out_shape=jax.ShapeDtypeStruct((1, 128), jnp.float32),
        interpret=interpret,
    )(nf, g, wn1, win1, bn1, wn2, win2, wgn2, bn2, *a1s, *cnt_args, *p_args)


def _head_body(*refs):
    (nsum_ref, g_ref, wgn_ref, wge_ref, wgg_ref, bg_ref,
     wm_ref, bm_ref, wl_ref, bl_ref) = refs[:10]
    es_refs = refs[10:10 + _K]
    mean_ref, logstd_ref = refs[-2:]
    esum = es_refs[0][...]
    for r in es_refs[1:]:
        esum = esum + r[...]
    u = (nsum_ref[...] * (1.0 / _N_NODES)) @ wgn_ref[...] \
        + (esum * (1.0 / _N_EDGES)) @ wge_ref[...] \
        + g_ref[...] @ wgg_ref[...] + bg_ref[...]
    gv = jnp.maximum(u, 0.0)
    mean_ref[...] = gv @ wm_ref[...] + bm_ref[...]
    logstd_ref[...] = jnp.clip(gv @ wl_ref[...] + bl_ref[...], -20.0, 2.0)


def _head_pass(nsum, g, wgn, wge, wgg, bg, wm, bm, wl, bl, esums,
               *, interpret=False):
    return pl.pallas_call(
        _head_body,
        out_shape=[
            jax.ShapeDtypeStruct((1, 8), jnp.float32),
            jax.ShapeDtypeStruct((1, 8), jnp.float32),
        ],
        interpret=interpret,
    )(nsum, g, wgn, wge, wgg, bg, wm, bm, wl, bl, *esums)


def kernel(node_features, edge_features, global_features, edge_index,
           W_e1, b_e1, W_n1, W_in1, b_n1,
           W_e2, W_ge2, b_e2,
           W_n2, W_in2, W_gn2, b_n2,
           W_gn, W_gedge, W_gg, b_g,
           W_mean, b_mean, W_logstd, b_logstd):
    n_pad_e = _N_EPAD - _N_EDGES
    recv = edge_index[1].astype(jnp.int32)
    pad_idx = _N_NODES + (jnp.arange(n_pad_e, dtype=jnp.int32)
                          % (_N_PAD - _N_NODES))
    recv_pad = jnp.concatenate([recv, pad_idx])
    zeros_n = jnp.zeros((_N_PAD, 128), jnp.float32)
    zeros1 = jnp.zeros((_N_PAD,), jnp.float32)
    ones_h = jnp.ones((_C,), jnp.float32)
    be1 = b_e1.reshape(1, -1)
    be2 = b_e2.reshape(1, -1)

    a1s, cnts, ps, esums = [], [], [], []
    for h in range(_K):
        recv_h = lax.slice_in_dim(recv_pad, h * _N_SL,
                                  (h + 1) * _N_SL).reshape(_N_CH, _C)
        e1, e2, esum = _edge_pass(
            edge_features, global_features, W_e1, be1, W_e2, W_ge2, be2,
            jnp.array([h], jnp.int32), h)
        agg1s, cnt_flat, agg2p = _sc_aggregate(
            e1, e2, recv_h, zeros_n, zeros1, ones_h)
        a1s.append(agg1s)
        cnts.append(cnt_flat.reshape(2, _N_PAD, 1))
        ps.append(agg2p)
        esums.append(esum)

    nsum = _node_pass(
        node_features, global_features, W_n1, W_in1, b_n1.reshape(1, -1),
        W_n2, W_in2, W_gn2, b_n2.reshape(1, -1), a1s, cnts, ps)
    return _head_pass(
        nsum, global_features, W_gn, W_gedge, W_gg, b_g.reshape(1, -1),
        W_mean, b_mean, W_logstd, b_logstd.reshape(1, -1), esums)


# speedup vs baseline: 1.1167x; 1.0009x over previous
import jax
import jax.numpy as jnp
from jax import lax
from jax.experimental import pallas as pl
from jax.experimental.pallas import tpu as pltpu
from jax.experimental.pallas import tpu_sc as plsc

_N_NODES = 10000
_N_EDGES = 320000
_C = 128
_N_EPAD = 327680
_K = 1
_N_SL = _N_EPAD // _K
_N_CH = _N_SL // _C
_B_E = 2560
_EB_TOT = _N_EDGES // _B_E
_EB_SL = _N_SL // _B_E
_B_N = 2000
_TILES = 16
_N_PAD = 10240
_ROWS_PT = _N_PAD // _TILES
_CH_A_PT = _N_CH // _TILES
_IDXB = 80
_IDX_BLKS = _CH_A_PT // _IDXB


def _edge_body(ef_ref, g_ref, we1_ref, be1_ref, we2_ref, wge2_ref, be2_ref,
               sl_ref, e1_ref, e2_ref, esum_ref):
    e1 = jnp.maximum(ef_ref[...] @ we1_ref[...] + be1_ref[...], 0.0)
    gterm = g_ref[...] @ wge2_ref[...] + be2_ref[...]
    e2 = jnp.maximum(e1 @ we2_ref[...] + gterm, 0.0)
    e1_ref[...] = e1
    e2_ref[...] = e2

    @pl.when(pl.program_id(0) == 0)
    def _():
        esum_ref[...] = jnp.zeros_like(esum_ref)

    rid = (sl_ref[0] * _N_SL + pl.program_id(0) * _B_E
           + lax.broadcasted_iota(jnp.int32, (_B_E, 1), 0))
    esum_ref[...] += jnp.sum(jnp.where(rid < _N_EDGES, e2, 0.0),
                             axis=0, keepdims=True)


def _edge_pass(ef, g, we1, be1, we2, wge2, be2, sl, h, *, interpret=False):
    base = h * _EB_SL
    return pl.pallas_call(
        _edge_body,
        grid=(_EB_SL,),
        in_specs=[
            pl.BlockSpec((_B_E, 16),
                         lambda i: (jnp.minimum(base + i, _EB_TOT - 1), 0)),
            pl.BlockSpec((1, 32), lambda i: (0, 0)),
            pl.BlockSpec((16, 256), lambda i: (0, 0)),
            pl.BlockSpec((1, 256), lambda i: (0, 0)),
            pl.BlockSpec((256, 128), lambda i: (0, 0)),
            pl.BlockSpec((32, 128), lambda i: (0, 0)),
            pl.BlockSpec((1, 128), lambda i: (0, 0)),
            pl.BlockSpec(memory_space=pltpu.SMEM),
        ],
        out_specs=[
            pl.BlockSpec((_B_E, 256), lambda i: (i, 0)),
            pl.BlockSpec((_B_E, 128), lambda i: (i, 0)),
            pl.BlockSpec((1, 128), lambda i: (0, 0)),
        ],
        out_shape=[
            jax.ShapeDtypeStruct((_N_SL, 256), jnp.float32),
            jax.ShapeDtypeStruct((_N_SL, 128), jnp.float32),
            jax.ShapeDtypeStruct((1, 128), jnp.float32),
        ],
        interpret=interpret,
    )(ef, g, we1, be1, we2, wge2, be2, sl)


def _sc_agg_body(e1_hbm, e2_hbm, recv_hbm, zeros_hbm, zeros1_hbm, ones_hbm,
                 agg1_out, cnt_out, agg2_out,
                 acc, cntacc, upd, idxb, ones_v, sem0, sem1):
    cid = lax.axis_index("c")
    sid = lax.axis_index("s")
    r0 = sid * _ROWS_PT
    col0 = cid * 128

    pltpu.sync_copy(zeros_hbm.at[pl.ds(r0, _ROWS_PT), :],
                    acc.at[pl.ds(r0, _ROWS_PT), :])
    pltpu.sync_copy(zeros1_hbm.at[pl.ds(r0, _ROWS_PT)],
                    cntacc.at[pl.ds(r0, _ROWS_PT)])
    pltpu.sync_copy(ones_hbm, ones_v)
    plsc.subcore_barrier()

    def _run_phase(src_slice, n_chunks, rowfn, with_counts):
        for blk in range(_IDX_BLKS):
            row0 = sid * _CH_A_PT + blk * _IDXB

            def _start(r, b):
                pltpu.async_copy(src_slice(row0 + r), upd.at[b],
                                 sem0 if b == 0 else sem1)

            def _wait(b):
                pltpu.make_async_copy(src_slice(0), upd.at[b],
                                      sem0 if b == 0 else sem1).wait()

            def _scatter(r, b):
                pltpu.sync_copy(upd.at[b], acc.at[idxb.at[r]], add=True)
                if with_counts:
                    pltpu.sync_copy(ones_v, cntacc.at[idxb.at[r]], add=True)

            pltpu.sync_copy(recv_hbm.at[pl.ds(row0, _IDXB), :], idxb)
            _start(rowfn(0), 0)

            def _pair(i, carry):
                k0 = 2 * i
                k1 = k0 + 1
                _start(rowfn(k1), 1)
                _wait(0)
                _scatter(rowfn(k0), 0)

                @pl.when(k0 + 2 < n_chunks)
                def _():
                    _start(rowfn(k0 + 2), 0)

                _wait(1)
                _scatter(rowfn(k1), 1)
                return carry

            lax.fori_loop(0, n_chunks // 2, _pair, 0)

    _run_phase(
        lambda ch: e1_hbm.at[pl.ds(ch * _C, _C), pl.ds(col0, 128)],
        _IDXB, lambda k: k, False)
    plsc.subcore_barrier()

    pltpu.sync_copy(acc.at[pl.ds(r0, _ROWS_PT), :],
                    agg1_out.at[pl.ds(r0, _ROWS_PT), pl.ds(col0, 128)])
    pltpu.sync_copy(zeros_hbm.at[pl.ds(r0, _ROWS_PT), :],
                    acc.at[pl.ds(r0, _ROWS_PT), :])
    plsc.subcore_barrier()

    _run_phase(
        lambda ch: e2_hbm.at[pl.ds(ch * _C, _C), :],
        _IDXB // 2, lambda k: 2 * k + cid, True)
    plsc.subcore_barrier()
    pltpu.sync_copy(acc.at[pl.ds(r0, _ROWS_PT), :],
                    agg2_out.at[cid, pl.ds(r0, _ROWS_PT), :])
    pltpu.sync_copy(cntacc.at[pl.ds(r0, _ROWS_PT)],
                    cnt_out.at[pl.ds(cid * _N_PAD + r0, _ROWS_PT)])


def _sc_aggregate(e1, e2, recv2, zeros_n, zeros1, ones_h):
    agg = pl.kernel(
        _sc_agg_body,
        cost_estimate=pl.CostEstimate(
            flops=0, bytes_accessed=1_000_000_000 // _K, transcendentals=0),
        out_type=[
            jax.ShapeDtypeStruct((_N_PAD, 256), jnp.float32),
            jax.ShapeDtypeStruct((2 * _N_PAD,), jnp.float32),
            jax.ShapeDtypeStruct((2, _N_PAD, 128), jnp.float32),
        ],
        mesh=plsc.VectorSubcoreMesh(core_axis_name="c", subcore_axis_name="s"),
        scratch_types=[
            pltpu.VMEM_SHARED((_N_PAD, 128), jnp.float32),
            pltpu.VMEM_SHARED((_N_PAD,), jnp.float32),
            pltpu.VMEM((2, _C, 128), jnp.float32),
            pltpu.VMEM((_IDXB, _C), jnp.int32),
            pltpu.VMEM((_C,), jnp.float32),
            pltpu.SemaphoreType.DMA,
            pltpu.SemaphoreType.DMA,
        ],
    )
    return agg(e1, e2, recv2, zeros_n, zeros1, ones_h)


def _node_body(*refs):
    (nf_ref, g_ref, wn1_ref, win1_ref, bn1_ref, wn2_ref, win2_ref,
     wgn2_ref, bn2_ref) = refs[:9]
    a1_refs = refs[9:9 + _K]
    cnt_refs = refs[9 + _K:9 + 3 * _K]
    p_refs = refs[9 + 3 * _K:9 + 5 * _K]
    nsum_ref = refs[-1]


    cnt = cnt_refs[0][0]
    for r in cnt_refs[1:]:
        cnt = cnt + r[0]
    cnt = jnp.maximum(cnt, 1.0)
    a1 = a1_refs[0][...]
    for r in a1_refs[1:]:
        a1 = a1 + r[...]
    p = p_refs[0][0]
    for r in p_refs[1:]:
        p = p + r[0]
    agg1 = a1 / cnt
    agg2 = p / cnt
    n1 = jnp.maximum(nf_ref[...] @ wn1_ref[...] + agg1 @ win1_ref[...]
                     + bn1_ref[...], 0.0)
    gterm = g_ref[...] @ wgn2_ref[...] + bn2_ref[...]
    n2 = jnp.maximum(n1 @ wn2_ref[...] + agg2 @ win2_ref[...] + gterm, 0.0)

    @pl.when(pl.program_id(0) == 0)
    def _():
        nsum_ref[...] = jnp.zeros_like(nsum_ref)

    nsum_ref[...] += jnp.sum(n2, axis=0, keepdims=True)

    @pl.when(pl.program_id(0) == pl.num_programs(0) - 1)
    def _():
        (wgn_ref, wge_ref, wgg_ref, bg_ref, wm_ref, bm_ref, wl_ref,
         bl_ref) = refs[9 + 5 * _K:17 + 5 * _K]
        es_refs = refs[17 + 5 * _K:17 + 6 * _K]
        mean_ref, logstd_ref = refs[-3:-1]
        esum = es_refs[0][...]
        for r in es_refs[1:]:
            esum = esum + r[...]
        u = (nsum_ref[...] * (1.0 / _N_NODES)) @ wgn_ref[...] \
            + (esum * (1.0 / _N_EDGES)) @ wge_ref[...] \
            + g_ref[...] @ wgg_ref[...] + bg_ref[...]
        gv = jnp.maximum(u, 0.0)
        mean_ref[...] = gv @ wm_ref[...] + bm_ref[...]
        logstd_ref[...] = jnp.clip(gv @ wl_ref[...] + bl_ref[...],
                                   -20.0, 2.0)


def _node_pass(nf, g, wn1, win1, bn1, wn2, win2, wgn2, bn2,
               a1s, cnts, ps, wgn, wge, wgg, bg, wm, bm, wl, bl, esums,
               *, interpret=False):
    n_blk = _N_NODES // _B_N
    node_blk = lambda w: pl.BlockSpec((_B_N, w), lambda i: (i, 0))
    const_blk = lambda r, c: pl.BlockSpec((r, c), lambda i: (0, 0))
    core_blk = lambda c, w: pl.BlockSpec((1, _B_N, w),
                                         lambda i, c=c: (c, i, 0))
    cnt_specs, cnt_args = [], []
    for arr in cnts:
        for c in range(2):
            cnt_specs.append(core_blk(c, 1))
            cnt_args.append(arr)
    p_specs, p_args = [], []
    for arr in ps:
        for c in range(2):
            p_specs.append(core_blk(c, 128))
            p_args.append(arr)
    return pl.pallas_call(
        _node_body,
        grid=(n_blk,),
        in_specs=(
            [node_blk(128), const_blk(1, 32),
             const_blk(128, 256), const_blk(256, 256), const_blk(1, 256),
             const_blk(256, 128), const_blk(128, 128), const_blk(32, 128),
             const_blk(1, 128)]
            + [node_blk(256)] * _K
            + cnt_specs + p_specs
            + [const_blk(128, 64), const_blk(128, 64), const_blk(32, 64),
               const_blk(1, 64), const_blk(64, 8), const_blk(1, 8),
               const_blk(64, 8), const_blk(1, 8)]
            + [const_blk(1, 128)] * _K
        ),
        out_specs=[pl.BlockSpec((1, 8), lambda i: (0, 0)),
                   pl.BlockSpec((1, 8), lambda i: (0, 0)),
                   pl.BlockSpec((1, 128), lambda i: (0, 0))],
        out_shape=[jax.ShapeDtypeStruct((1, 8), jnp.float32),
                   jax.ShapeDtypeStruct((1, 8), jnp.float32),
                   jax.ShapeDtypeStruct((1, 128), jnp.float32)],
        interpret=interpret,
    )(nf, g, wn1, win1, bn1, wn2, win2, wgn2, bn2, *a1s, *cnt_args, *p_args,
      wgn, wge, wgg, bg, wm, bm, wl, bl, *esums)


def _head_body(*refs):
    (nsum_ref, g_ref, wgn_ref, wge_ref, wgg_ref, bg_ref,
     wm_ref, bm_ref, wl_ref, bl_ref) = refs[:10]
    es_refs = refs[10:10 + _K]
    mean_ref, logstd_ref = refs[-2:]
    esum = es_refs[0][...]
    for r in es_refs[1:]:
        esum = esum + r[...]
    u = (nsum_ref[...] * (1.0 / _N_NODES)) @ wgn_ref[...] \
        + (esum * (1.0 / _N_EDGES)) @ wge_ref[...] \
        + g_ref[...] @ wgg_ref[...] + bg_ref[...]
    gv = jnp.maximum(u, 0.0)
    mean_ref[...] = gv @ wm_ref[...] + bm_ref[...]
    logstd_ref[...] = jnp.clip(gv @ wl_ref[...] + bl_ref[...], -20.0, 2.0)


def _head_pass(nsum, g, wgn, wge, wgg, bg, wm, bm, wl, bl, esums,
               *, interpret=False):
    return pl.pallas_call(
        _head_body,
        out_shape=[
            jax.ShapeDtypeStruct((1, 8), jnp.float32),
            jax.ShapeDtypeStruct((1, 8), jnp.float32),
        ],
        interpret=interpret,
    )(nsum, g, wgn, wge, wgg, bg, wm, bm, wl, bl, *esums)


def kernel(node_features, edge_features, global_features, edge_index,
           W_e1, b_e1, W_n1, W_in1, b_n1,
           W_e2, W_ge2, b_e2,
           W_n2, W_in2, W_gn2, b_n2,
           W_gn, W_gedge, W_gg, b_g,
           W_mean, b_mean, W_logstd, b_logstd):
    n_pad_e = _N_EPAD - _N_EDGES
    recv = edge_index[1].astype(jnp.int32)
    pad_idx = _N_NODES + (jnp.arange(n_pad_e, dtype=jnp.int32)
                          % (_N_PAD - _N_NODES))
    recv_pad = jnp.concatenate([recv, pad_idx])
    zeros_n = jnp.zeros((_N_PAD, 128), jnp.float32)
    zeros1 = jnp.zeros((_N_PAD,), jnp.float32)
    ones_h = jnp.ones((_C,), jnp.float32)
    be1 = b_e1.reshape(1, -1)
    be2 = b_e2.reshape(1, -1)

    a1s, cnts, ps, esums = [], [], [], []
    for h in range(_K):
        recv_h = lax.slice_in_dim(recv_pad, h * _N_SL,
                                  (h + 1) * _N_SL).reshape(_N_CH, _C)
        e1, e2, esum = _edge_pass(
            edge_features, global_features, W_e1, be1, W_e2, W_ge2, be2,
            jnp.array([h], jnp.int32), h)
        agg1s, cnt_flat, agg2p = _sc_aggregate(
            e1, e2, recv_h, zeros_n, zeros1, ones_h)
        a1s.append(agg1s)
        cnts.append(cnt_flat.reshape(2, _N_PAD, 1))
        ps.append(agg2p)
        esums.append(esum)

    mean, log_std, _ = _node_pass(
        node_features, global_features, W_n1, W_in1, b_n1.reshape(1, -1),
        W_n2, W_in2, W_gn2, b_n2.reshape(1, -1), a1s, cnts, ps,
        W_gn, W_gedge, W_gg, b_g.reshape(1, -1),
        W_mean, b_mean.reshape(1, -1), W_logstd, b_logstd.reshape(1, -1),
        esums)
    return (mean, log_std)
